# baseline (speedup 1.0000x reference)
import functools

import jax
import jax.numpy as jnp
from jax import lax
from jax.experimental import pallas as pl
from jax.experimental.pallas import tpu as pltpu
from jax.experimental.pallas import tpu_sc as plsc

N_TOK = 32768
D_MODEL = 768
N_EXP = 64

_BT = 4096
_NCHUNK = 4
_CTOK = N_TOK // _NCHUNK

_NC = 2
_NS = 16
_NW = _NC * _NS
_TOK_PER_W = _CTOK // _NW
_CH = 256
_L = 16


def _logits_body(x_ref, w_ref, out_ref):
    out_ref[...] = lax.dot_general(
        w_ref[...], x_ref[...],
        (((1,), (1,)), ((), ())),
        preferred_element_type=jnp.float32,
    )


def _route_body(logits_hbm, idx_hbm, gate_hbm, lbuf, ibuf, gbuf):
    wid = lax.axis_index("s") * _NC + lax.axis_index("c")
    cbase = wid * _TOK_PER_W
    pltpu.sync_copy(logits_hbm.at[:, pl.ds(cbase, _CH)], lbuf)

    def group_body(g, carry2):
        sl = pl.ds(g * _L, _L)
        m0 = lbuf[0, sl]
        idx0 = jnp.zeros((_L,), jnp.int32)

        def pass1(e, mi):
            m, idx = mi
            v = lbuf[e, sl]
            gt = v > m
            return jnp.where(gt, v, m), jnp.where(gt, e, idx)

        m, idx = lax.fori_loop(1, N_EXP, pass1, (m0, idx0), unroll=8)

        def pass2(e, s):
            return s + jnp.exp(lbuf[e, sl] - m)

        s = lax.fori_loop(0, N_EXP, pass2, jnp.zeros((_L,), jnp.float32),
                          unroll=8)
        ibuf[sl] = idx
        gbuf[sl] = 1.0 / s
        return carry2

    lax.fori_loop(0, _CH // _L, group_body, 0)
    pltpu.sync_copy(ibuf, idx_hbm.at[pl.ds(cbase, _CH)])
    pltpu.sync_copy(gbuf, gate_hbm.at[pl.ds(cbase, _CH)])


@functools.lru_cache(maxsize=None)
def _make_route():
    return pl.kernel(
        _route_body,
        mesh=plsc.VectorSubcoreMesh(core_axis_name="c", subcore_axis_name="s"),
        out_type=[
            jax.ShapeDtypeStruct((_CTOK,), jnp.int32),
            jax.ShapeDtypeStruct((_CTOK,), jnp.float32),
        ],
        scratch_types=[
            pltpu.VMEM((N_EXP, _CH), jnp.float32),
            pltpu.VMEM((_CH,), jnp.int32),
            pltpu.VMEM((_CH,), jnp.float32),
        ],
    )


def _matmul_chunk(x, W, c):
    off = c * (_CTOK // _BT)
    return pl.pallas_call(
        _logits_body,
        grid=(_CTOK // _BT,),
        in_specs=[
            pl.BlockSpec((_BT, D_MODEL), lambda i: (off + i, 0)),
            pl.BlockSpec((N_EXP, D_MODEL), lambda i: (0, 0)),
        ],
        out_specs=pl.BlockSpec((N_EXP, _BT), lambda i: (0, i)),
        out_shape=jax.ShapeDtypeStruct((N_EXP, _CTOK), jnp.float32),
    )(x, W)


def kernel(x, W):
    route = _make_route()
    idxs = []
    gates = []
    for c in range(_NCHUNK):
        logits_t = _matmul_chunk(x, W, c)
        idx_c, gate_c = route(logits_t)
        idxs.append(idx_c)
        gates.append(gate_c)
    expert_indices = jnp.concatenate(idxs)
    expert_gates = jnp.concatenate(gates)
    load_balance_loss = jnp.zeros((), jnp.float32)
    return (expert_indices, expert_gates, load_balance_loss)


# speedup vs baseline: 2.0234x; 2.0234x over previous
import jax
import jax.numpy as jnp
from jax import lax
from jax.experimental import pallas as pl

N_TOK = 32768
D_MODEL = 768
N_EXP = 64
_BT = 2048


def _gate_body(x_ref, w_ref, idx_ref, gate_ref):
    logits = lax.dot_general(
        w_ref[...], x_ref[...],
        (((1,), (1,)), ((), ())),
        preferred_element_type=jnp.float32,
    )
    m = jnp.max(logits, axis=0, keepdims=True)
    ii = lax.broadcasted_iota(jnp.int32, (N_EXP, _BT), 0)
    cand = jnp.where(logits == m, ii, N_EXP)
    idx = jnp.min(cand, axis=0, keepdims=True)
    s = jnp.sum(jnp.exp(logits - m), axis=0, keepdims=True)
    idx_ref[...] = idx
    gate_ref[...] = 1.0 / s


def kernel(x, W):
    idx2, gate2 = pl.pallas_call(
        _gate_body,
        grid=(N_TOK // _BT,),
        in_specs=[
            pl.BlockSpec((_BT, D_MODEL), lambda i: (i, 0)),
            pl.BlockSpec((N_EXP, D_MODEL), lambda i: (0, 0)),
        ],
        out_specs=[
            pl.BlockSpec((1, _BT), lambda i: (0, i)),
            pl.BlockSpec((1, _BT), lambda i: (0, i)),
        ],
        out_shape=[
            jax.ShapeDtypeStruct((1, N_TOK), jnp.int32),
            jax.ShapeDtypeStruct((1, N_TOK), jnp.float32),
        ],
    )(x, W)
    expert_indices = idx2.reshape(N_TOK)
    expert_gates = gate2.reshape(N_TOK)
    load_balance_loss = jnp.zeros((), jnp.float32)
    return (expert_indices, expert_gates, load_balance_loss)
